# baseline (speedup 1.0000x reference)
import functools

import jax
import jax.numpy as jnp
from jax import lax
from jax.experimental import pallas as pl
from jax.experimental.pallas import tpu as pltpu
from jax.experimental.pallas import tpu_sc as plsc

N = 8192
D = 512
LAM = 0.5
NC = 2
NS = 16
NW = NC * NS
RPW = N // NW
CH = 64
NCHUNK = RPW // CH
KV = D // 16
G = 16
NG = CH // G

_MESH = plsc.VectorSubcoreMesh(
    core_axis_name="c", subcore_axis_name="s", num_cores=NC, num_subcores=NS
)

_GATHER_DNUMS = lax.GatherDimensionNumbers(
    offset_dims=(), collapsed_slice_dims=(0,), start_index_map=(0,)
)


def _shuffle(v, idx):
    return lax.gather(
        v, idx[:, None], _GATHER_DNUMS, slice_sizes=(1,),
        mode=lax.GatherScatterMode.PROMISE_IN_BOUNDS,
    )


def _allsum(v):
    lane = lax.iota(jnp.int32, 16)
    for m in (8, 4, 2, 1):
        v = v + _shuffle(v, lane ^ m)
    return v


def _lane0_mask():
    return lax.iota(jnp.int32, 16) == 0


def _rot_up_idx():
    lane = lax.iota(jnp.int32, 16)
    return (lane + 15) & 15


def _rot_insert(vec, val, mask0, rotidx):
    return jnp.where(mask0, val, _shuffle(vec, rotidx))


def _rsqrt_newton(n2):
    i = lax.bitcast_convert_type(n2, jnp.int32)
    i = jnp.int32(0x5F3759DF) - (i >> 1)
    y = lax.bitcast_convert_type(i, jnp.float32)
    for _ in range(3):
        y = y * (1.5 - 0.5 * n2 * y * y)
    return y


def _chunk_loop(x_hbm, base, bufs, sems, process):
    for b in range(2):
        pltpu.async_copy(x_hbm.at[pl.ds(base + b * CH, CH)], bufs[b], sems[b])

    @pl.loop(0, NCHUNK, step=2)
    def chunk_pair(cp):
        for b in range(2):
            ch = cp + b
            pltpu.make_async_copy(x_hbm.at[pl.ds(0, CH)], bufs[b], sems[b]).wait()
            process(bufs[b], ch)

            @pl.when(ch + 2 < NCHUNK)
            def _():
                pltpu.async_copy(
                    x_hbm.at[pl.ds(base + (ch + 2) * CH, CH)], bufs[b], sems[b]
                )


@functools.partial(
    pl.kernel,
    out_type=jax.ShapeDtypeStruct((NW, D), jnp.float32),
    mesh=_MESH,
    scratch_types=[
        pltpu.VMEM((CH, D), jnp.float32),
        pltpu.VMEM((CH, D), jnp.float32),
        pltpu.VMEM((D,), jnp.float32),
        pltpu.VMEM((CH,), jnp.float32),
        pltpu.SemaphoreType.DMA,
        pltpu.SemaphoreType.DMA,
    ],
)
def _pass1(x_hbm, s_out, bufa, bufb, s_acc, r_chunk, sema, semb):
    wid = lax.axis_index("s") * NC + lax.axis_index("c")
    base = wid * RPW
    zero16 = jnp.zeros((16,), jnp.float32)
    for k in range(KV):
        s_acc[pl.ds(k * 16, 16)] = zero16

    def process(cur, ch):
        @plsc.parallel_loop(0, NG, unroll=2)
        def norm_group(g):
            mask0 = _lane0_mask()
            rotidx = _rot_up_idx()
            n2vec = zero16
            for j in range(G):
                i = g * G + j
                a0 = a1 = a2 = a3 = zero16
                for k in range(0, KV, 4):
                    v0 = cur[i, pl.ds(k * 16, 16)]
                    v1 = cur[i, pl.ds((k + 1) * 16, 16)]
                    v2 = cur[i, pl.ds((k + 2) * 16, 16)]
                    v3 = cur[i, pl.ds((k + 3) * 16, 16)]
                    a0 = a0 + v0 * v0
                    a1 = a1 + v1 * v1
                    a2 = a2 + v2 * v2
                    a3 = a3 + v3 * v3
                n2 = _allsum((a0 + a1) + (a2 + a3))
                n2vec = _rot_insert(n2vec, n2, mask0, rotidx)
            r_chunk[pl.ds(g * G, 16)] = _rsqrt_newton(lax.rev(n2vec, (0,)))

        zeroidx = jnp.zeros((16,), jnp.int32)
        rotdn = (lax.iota(jnp.int32, 16) + 1) & 15

        def grp_body(g, sacc):
            rot = r_chunk[pl.ds(g * G, 16)]
            out = list(sacc)
            for j in range(G):
                rb = _shuffle(rot, zeroidx)
                rot = _shuffle(rot, rotdn)
                i = g * G + j
                for k in range(KV):
                    out[k] = out[k] + cur[i, pl.ds(k * 16, 16)] * rb
            return tuple(out)

        sacc = tuple(s_acc[pl.ds(k * 16, 16)] for k in range(KV))
        sacc = lax.fori_loop(0, NG, grp_body, sacc)
        for k in range(KV):
            s_acc[pl.ds(k * 16, 16)] = sacc[k]

    _chunk_loop(x_hbm, base, (bufa, bufb), (sema, semb), process)

    pltpu.sync_copy(s_acc, s_out.at[wid])


@functools.partial(
    pl.kernel,
    out_type=jax.ShapeDtypeStruct((N,), jnp.float32),
    mesh=_MESH,
    scratch_types=[
        pltpu.VMEM((CH, D), jnp.float32),
        pltpu.VMEM((CH, D), jnp.float32),
        pltpu.VMEM((NW, D), jnp.float32),
        pltpu.VMEM((RPW,), jnp.float32),
        pltpu.SemaphoreType.DMA,
        pltpu.SemaphoreType.DMA,
    ],
)
def _pass2(x_hbm, spart_hbm, out_hbm, bufa, bufb, spart_v, out_buf, sema, semb):
    wid = lax.axis_index("s") * NC + lax.axis_index("c")
    base = wid * RPW
    lane = lax.iota(jnp.int32, 16)
    zero16 = jnp.zeros((16,), jnp.float32)
    pltpu.sync_copy(spart_hbm, spart_v)

    def red_body(w, acc):
        return tuple(
            acc[k] + spart_v[w, pl.ds(k * 16, 16)] for k in range(KV)
        )

    svals = lax.fori_loop(
        1, NW, red_body, tuple(spart_v[0, pl.ds(k * 16, 16)] for k in range(KV))
    )

    def process(cur, ch):
        @plsc.parallel_loop(0, NG, unroll=2)
        def gains_group(g):
            mask0 = _lane0_mask()
            rotidx = _rot_up_idx()
            n2vec = zero16
            dotvec = zero16
            for j in range(G):
                i = g * G + j
                d0 = d1 = q0 = q1 = zero16
                for k in range(0, KV, 2):
                    v0 = cur[i, pl.ds(k * 16, 16)]
                    v1 = cur[i, pl.ds((k + 1) * 16, 16)]
                    d0 = d0 + v0 * svals[k]
                    d1 = d1 + v1 * svals[k + 1]
                    q0 = q0 + v0 * v0
                    q1 = q1 + v1 * v1
                n2 = _allsum(q0 + q1)
                dot = _allsum(d0 + d1)
                n2vec = _rot_insert(n2vec, n2, mask0, rotidx)
                dotvec = _rot_insert(dotvec, dot, mask0, rotidx)
            gvec = lax.rev(dotvec, (0,)) * _rsqrt_newton(lax.rev(n2vec, (0,))) - LAM
            out_buf[pl.ds(ch * CH + g * G, 16)] = gvec

    _chunk_loop(x_hbm, base, (bufa, bufb), (sema, semb), process)

    pltpu.sync_copy(out_buf, out_hbm.at[pl.ds(base, RPW)])


def kernel(X):
    s_part = _pass1(X)
    return _pass2(X, s_part)


# speedup vs baseline: 1.4051x; 1.4051x over previous
import functools

import jax
import jax.numpy as jnp
from jax import lax
from jax.experimental import pallas as pl
from jax.experimental.pallas import tpu as pltpu
from jax.experimental.pallas import tpu_sc as plsc

N = 8192
D = 512
LAM = 0.5
NC = 2
NS = 16
NW = NC * NS
RPW = N // NW
CH = 64
NCHUNK = RPW // CH
KV = D // 16
G = 16
NG = CH // G

_MESH = plsc.VectorSubcoreMesh(
    core_axis_name="c", subcore_axis_name="s", num_cores=NC, num_subcores=NS
)

_GATHER_DNUMS = lax.GatherDimensionNumbers(
    offset_dims=(), collapsed_slice_dims=(0,), start_index_map=(0,)
)


def _shuffle(v, idx):
    return lax.gather(
        v, idx[:, None], _GATHER_DNUMS, slice_sizes=(1,),
        mode=lax.GatherScatterMode.PROMISE_IN_BOUNDS,
    )


def _allsum(v):
    lane = lax.iota(jnp.int32, 16)
    for m in (8, 4, 2, 1):
        v = v + _shuffle(v, lane ^ m)
    return v


def _lane0_mask():
    return lax.iota(jnp.int32, 16) == 0


def _rot_up_idx():
    lane = lax.iota(jnp.int32, 16)
    return (lane + 15) & 15


def _rot_insert(vec, val, mask0, rotidx):
    return jnp.where(mask0, val, _shuffle(vec, rotidx))


def _rsqrt_newton(n2):
    i = lax.bitcast_convert_type(n2, jnp.int32)
    i = jnp.int32(0x5F3759DF) - (i >> 1)
    y = lax.bitcast_convert_type(i, jnp.float32)
    for _ in range(3):
        y = y * (1.5 - 0.5 * n2 * y * y)
    return y


def _chunk_loop(x_hbm, base, bufs, sems, process):
    for b in range(2):
        pltpu.async_copy(x_hbm.at[pl.ds(base + b * CH, CH)], bufs[b], sems[b])

    @pl.loop(0, NCHUNK, step=2)
    def chunk_pair(cp):
        for b in range(2):
            ch = cp + b
            pltpu.make_async_copy(x_hbm.at[pl.ds(0, CH)], bufs[b], sems[b]).wait()
            process(bufs[b], ch)

            @pl.when(ch + 2 < NCHUNK)
            def _():
                pltpu.async_copy(
                    x_hbm.at[pl.ds(base + (ch + 2) * CH, CH)], bufs[b], sems[b]
                )


@functools.partial(
    pl.kernel,
    out_type=jax.ShapeDtypeStruct((NW, D), jnp.float32),
    mesh=_MESH,
    scratch_types=[
        pltpu.VMEM((CH, D), jnp.float32),
        pltpu.VMEM((CH, D), jnp.float32),
        pltpu.VMEM((D,), jnp.float32),
        pltpu.VMEM((CH,), jnp.float32),
        pltpu.SemaphoreType.DMA,
        pltpu.SemaphoreType.DMA,
    ],
)
def _pass1(x_hbm, s_out, bufa, bufb, s_acc, r_chunk, sema, semb):
    wid = lax.axis_index("s") * NC + lax.axis_index("c")
    base = wid * RPW
    zero16 = jnp.zeros((16,), jnp.float32)
    for k in range(KV):
        s_acc[pl.ds(k * 16, 16)] = zero16

    def process(cur, ch):
        @plsc.parallel_loop(0, NG)
        def norm_group(g):
            mask0 = _lane0_mask()
            rotidx = _rot_up_idx()
            n2vec = zero16
            for j in range(G):
                i = g * G + j
                a0 = a1 = a2 = a3 = zero16
                for k in range(0, KV, 4):
                    v0 = cur[i, pl.ds(k * 16, 16)]
                    v1 = cur[i, pl.ds((k + 1) * 16, 16)]
                    v2 = cur[i, pl.ds((k + 2) * 16, 16)]
                    v3 = cur[i, pl.ds((k + 3) * 16, 16)]
                    a0 = a0 + v0 * v0
                    a1 = a1 + v1 * v1
                    a2 = a2 + v2 * v2
                    a3 = a3 + v3 * v3
                n2 = _allsum((a0 + a1) + (a2 + a3))
                n2vec = _rot_insert(n2vec, n2, mask0, rotidx)
            r_chunk[pl.ds(g * G, 16)] = _rsqrt_newton(lax.rev(n2vec, (0,)))

        zeroidx = jnp.zeros((16,), jnp.int32)
        rotdn = (lax.iota(jnp.int32, 16) + 1) & 15

        def grp_body(g, sacc):
            rot = r_chunk[pl.ds(g * G, 16)]
            out = list(sacc)
            for j in range(G):
                rb = _shuffle(rot, zeroidx)
                rot = _shuffle(rot, rotdn)
                i = g * G + j
                for k in range(KV):
                    out[k] = out[k] + cur[i, pl.ds(k * 16, 16)] * rb
            return tuple(out)

        sacc = tuple(s_acc[pl.ds(k * 16, 16)] for k in range(KV))
        sacc = lax.fori_loop(0, NG, grp_body, sacc)
        for k in range(KV):
            s_acc[pl.ds(k * 16, 16)] = sacc[k]

    _chunk_loop(x_hbm, base, (bufa, bufb), (sema, semb), process)

    pltpu.sync_copy(s_acc, s_out.at[wid])


@functools.partial(
    pl.kernel,
    out_type=jax.ShapeDtypeStruct((N,), jnp.float32),
    mesh=_MESH,
    scratch_types=[
        pltpu.VMEM((CH, D), jnp.float32),
        pltpu.VMEM((CH, D), jnp.float32),
        pltpu.VMEM((NW, D), jnp.float32),
        pltpu.VMEM((RPW,), jnp.float32),
        pltpu.SemaphoreType.DMA,
        pltpu.SemaphoreType.DMA,
    ],
)
def _pass2(x_hbm, spart_hbm, out_hbm, bufa, bufb, spart_v, out_buf, sema, semb):
    wid = lax.axis_index("s") * NC + lax.axis_index("c")
    base = wid * RPW
    lane = lax.iota(jnp.int32, 16)
    zero16 = jnp.zeros((16,), jnp.float32)
    pltpu.sync_copy(spart_hbm, spart_v)

    def red_body(w, acc):
        return tuple(
            acc[k] + spart_v[w, pl.ds(k * 16, 16)] for k in range(KV)
        )

    svals = lax.fori_loop(
        1, NW, red_body, tuple(spart_v[0, pl.ds(k * 16, 16)] for k in range(KV))
    )

    def process(cur, ch):
        @plsc.parallel_loop(0, NG)
        def gains_group(g):
            mask0 = _lane0_mask()
            rotidx = _rot_up_idx()
            n2vec = zero16
            dotvec = zero16
            for j in range(G):
                i = g * G + j
                d0 = d1 = q0 = q1 = zero16
                for k in range(0, KV, 2):
                    v0 = cur[i, pl.ds(k * 16, 16)]
                    v1 = cur[i, pl.ds((k + 1) * 16, 16)]
                    d0 = d0 + v0 * svals[k]
                    d1 = d1 + v1 * svals[k + 1]
                    q0 = q0 + v0 * v0
                    q1 = q1 + v1 * v1
                n2 = _allsum(q0 + q1)
                dot = _allsum(d0 + d1)
                n2vec = _rot_insert(n2vec, n2, mask0, rotidx)
                dotvec = _rot_insert(dotvec, dot, mask0, rotidx)
            gvec = lax.rev(dotvec, (0,)) * _rsqrt_newton(lax.rev(n2vec, (0,))) - LAM
            out_buf[pl.ds(ch * CH + g * G, 16)] = gvec

    _chunk_loop(x_hbm, base, (bufa, bufb), (sema, semb), process)

    pltpu.sync_copy(out_buf, out_hbm.at[pl.ds(base, RPW)])


def kernel(X):
    s_part = _pass1(X)
    return _pass2(X, s_part)
